# Initial kernel scaffold; baseline (speedup 1.0000x reference)
import jax
import jax.numpy as jnp
from jax.experimental import pallas as pl


def kernel(z, prior_samples):
    raise NotImplementedError("write your pallas kernel here")


# speedup vs baseline: 3.0885x; 3.0885x over previous
import functools

import jax
import jax.numpy as jnp
from jax import lax
from jax.experimental import pallas as pl
from jax.experimental.pallas import tpu as pltpu
from jax.experimental.pallas import tpu_sc as plsc

GROUP = 4
K = 1024
J = 16
LOGVAR_MIN, LOGVAR_MAX = -30.0, 20.0

ROW_BLOCK = 144

NW = 32
CHUNK = 128


def _score_body(z_ref, st_ref, idx_ref):
    zb = z_ref[...]
    R = zb.shape[0]
    acc = jnp.zeros((R, J, K), jnp.float32)
    for g in range(GROUP):
        mu_g = zb[:, g * J:(g + 1) * J]
        lv_g = jnp.clip(zb[:, 64 + g * J:64 + (g + 1) * J],
                        LOGVAR_MIN, LOGVAR_MAX)
        iv_g = jnp.exp(-lv_g)
        a_g = mu_g * iv_g
        c_g = 0.5 * (1.0 - iv_g)
        sg = st_ref[g, :]
        acc = (acc
               + a_g[:, :, None] * sg[None, None, :]
               + c_g[:, :, None] * (sg * sg)[None, None, :])
    m = jnp.max(acc, axis=2, keepdims=True)
    iot = lax.broadcasted_iota(jnp.int32, (R, J, K), 2)
    idx_ref[...] = jnp.min(jnp.where(acc >= m, iot, K), axis=2)


def _tc_indices(z2d, s_t, interpret=False):
    n_bl = z2d.shape[0]
    grid = n_bl // ROW_BLOCK
    return pl.pallas_call(
        _score_body,
        grid=(grid,),
        in_specs=[
            pl.BlockSpec((ROW_BLOCK, 128), lambda i: (i, 0)),
            pl.BlockSpec((GROUP, K), lambda i: (0, 0)),
        ],
        out_specs=pl.BlockSpec((ROW_BLOCK, J), lambda i: (i, 0)),
        out_shape=jax.ShapeDtypeStruct((n_bl, J), jnp.int32),
        interpret=interpret,
    )(z2d, s_t)


def _sc_gather(table_flat, idx_flat):
    n = idx_flat.shape[0]
    bpw = n // NW
    nvec = bpw // 16
    mesh = plsc.VectorSubcoreMesh(core_axis_name="c", subcore_axis_name="s")

    @functools.partial(
        pl.kernel,
        mesh=mesh,
        out_type=jax.ShapeDtypeStruct((NW, GROUP, bpw), jnp.float32),
        scratch_types=[
            pltpu.VMEM((K * GROUP,), jnp.float32),
            pltpu.VMEM((bpw,), jnp.int32),
            pltpu.VMEM((GROUP, bpw), jnp.float32),
        ],
        compiler_params=pltpu.CompilerParams(needs_layout_passes=False),
    )
    def gather_k(table_hbm, idx_hbm, out_hbm, tbl_v, idx_v, out_v):
        wid = lax.axis_index("s") * 2 + lax.axis_index("c")
        pltpu.sync_copy(table_hbm, tbl_v)
        pltpu.sync_copy(idx_hbm.at[pl.ds(wid * bpw, bpw)], idx_v)

        def body(i, _):
            off = i * 16
            lin = idx_v[pl.ds(off, 16)] * GROUP
            for g in range(GROUP):
                out_v[g, pl.ds(off, 16)] = plsc.load_gather(tbl_v, [lin + g])
            return _

        lax.fori_loop(0, nvec, body, None)
        pltpu.sync_copy(out_v, out_hbm.at[wid])

    return gather_k(table_flat, idx_flat)


def kernel(z, prior_samples):
    z = z.astype(jnp.float32)
    b, l, c2 = z.shape
    c = c2 // 2
    n = b * l * J

    z2d = z.reshape(b * l, c2)
    s_t = prior_samples.T

    idx2d = _tc_indices(z2d, s_t)
    indices = idx2d.reshape(b, l, c // GROUP)

    planes = _sc_gather(prior_samples.reshape(-1), idx2d.reshape(n))
    zhat_rows = planes.transpose(0, 2, 1).reshape(n, GROUP)
    zhat = (zhat_rows.reshape(b, l, c // GROUP, GROUP)
            .transpose(0, 1, 3, 2).reshape(b, l, c))
    return zhat, indices
